# baseline (speedup 1.0000x reference)
import functools

import jax
import jax.numpy as jnp
from jax import lax
from jax.experimental import pallas as pl
from jax.experimental.pallas import tpu as pltpu
from jax.experimental.pallas import tpu_sc as plsc

_B, _C, _H, _W = 4, 19, 512, 512
_HW = _H * _W
_NW = 32
_WPB = _NW // _B
_PPW = _HW // _WPB
_F = 1024
_NCH = _PPW // _F
_G = _F // 16

_LN2 = 0.6931471805599453
_EPS = 1e-3


def _ln(s):
    bits = lax.bitcast_convert_type(s, jnp.int32)
    y = bits.astype(jnp.float32) * (_LN2 / 8388608.0) - ((127.0 - 0.0450466) * _LN2)
    for _ in range(2):
        y = y + s * jnp.exp(-y) - 1.0
    return y


def _sc_body(score_hbm, target_hbm, out_hbm, sbuf, tbuf, obuf,
             ss0, ss1, st0, st1):
    cid = lax.axis_index("c")
    sid = lax.axis_index("s")
    wid = sid * 2 + cid
    b = wid // _WPB
    woff = (wid % _WPB) * _PPW
    sems = ((ss0, st0), (ss1, st1))

    def start(j, par):
        off = woff + j * _F
        pltpu.async_copy(score_hbm.at[b, :, pl.ds(off, _F)], sbuf.at[par],
                         sems[par][0])
        pltpu.async_copy(target_hbm.at[b, pl.ds(off, _F)], tbuf.at[par],
                         sems[par][1])

    def wait(par):
        pltpu.make_async_copy(score_hbm.at[b, :, pl.ds(woff, _F)],
                              sbuf.at[par], sems[par][0]).wait()
        pltpu.make_async_copy(target_hbm.at[b, pl.ds(woff, _F)],
                              tbuf.at[par], sems[par][1]).wait()

    def make_group(par):
        sref = sbuf.at[par]

        def group(i, accs):
            nll, cnt, saa, sbb, scc = accs
            base = i * 16
            t = tbuf[par, pl.ds(base, 16)]
            xs = [sbuf[par, c, pl.ds(base, 16)] for c in range(_C)]
            m = xs[0]
            am = jnp.zeros((16,), jnp.float32)
            for c in range(1, _C):
                gt = xs[c] > m
                m = jnp.where(gt, xs[c], m)
                am = jnp.where(gt, jnp.float32(c), am)
            s = jnp.exp(xs[0] - m)
            for c in range(1, _C):
                s = s + jnp.exp(xs[c] - m)
            pidx = base + lax.iota(jnp.int32, 16)
            t0 = jnp.maximum(t, 0)
            xt = plsc.load_gather(sref, [t0, pidx])
            lse = _ln(s) + m
            valid = t >= 0
            vf = jnp.where(valid, 1.0, 0.0).astype(jnp.float32)
            tf = t.astype(jnp.float32)
            nll = nll + jnp.where(valid, lse - xt, 0.0)
            cnt = cnt + vf
            saa = saa + am * tf
            sbb = sbb + am * am
            scc = scc + tf * tf
            return (nll, cnt, saa, sbb, scc)

        return group

    start(0, 0)
    start(1, 1)

    def pair(j2, accs):
        j = j2 * 2
        for par in range(2):
            jj = j + par
            wait(par)
            accs = lax.fori_loop(0, _G, make_group(par), accs, unroll=4)
            start(jnp.minimum(jj + 2, _NCH - 1), par)
        return accs

    zeros = jnp.zeros((16,), jnp.float32)
    accs = lax.fori_loop(0, _NCH // 2, pair,
                         (zeros, zeros, zeros, zeros, zeros), unroll=False)
    wait(0)
    wait(1)
    for q in range(5):
        obuf[q, :] = accs[q]
    pltpu.sync_copy(obuf, out_hbm.at[wid])


@jax.jit
def _run(score3, target2):
    mesh = plsc.VectorSubcoreMesh(core_axis_name="c", subcore_axis_name="s")
    call = pl.kernel(
        _sc_body,
        out_type=jax.ShapeDtypeStruct((_NW, 5, 16), jnp.float32),
        mesh=mesh,
        scratch_types=[
            pltpu.VMEM((2, _C, _F), jnp.float32),
            pltpu.VMEM((2, _F), jnp.int32),
            pltpu.VMEM((5, 16), jnp.float32),
            pltpu.SemaphoreType.DMA,
            pltpu.SemaphoreType.DMA,
            pltpu.SemaphoreType.DMA,
            pltpu.SemaphoreType.DMA,
        ],
        compiler_params=pltpu.CompilerParams(needs_layout_passes=False),
    )
    part = call(score3, target2)
    part = part.sum(axis=2)
    per_b = part.reshape(_B, _WPB, 5).sum(axis=1)
    nll_tot = per_b[:, 0].sum()
    cnt_tot = per_b[:, 1].sum()
    ce = nll_tot / jnp.maximum(cnt_tot, 1.0)
    a = per_b[:, 2]
    bb = per_b[:, 3] + _EPS
    cc = per_b[:, 4] + _EPS
    dice = 1.0 - 2.0 * a / (bb + cc)
    return ce + dice


def kernel(score, target, epoch):
    score3 = score.reshape(_B, _C, _HW)
    target2 = target.reshape(_B, _HW)
    return _run(score3, target2)


# speedup vs baseline: 1.5358x; 1.5358x over previous
import functools

import jax
import jax.numpy as jnp
from jax import lax
from jax.experimental import pallas as pl
from jax.experimental.pallas import tpu as pltpu
from jax.experimental.pallas import tpu_sc as plsc

_B, _C, _H, _W = 4, 19, 512, 512
_HW = _H * _W
_NW = 32
_WPB = _NW // _B
_PPW = _HW // _WPB
_F = 1024
_NCH = _PPW // _F
_G = _F // 16

_LN2 = 0.6931471805599453
_EPS = 1e-3


def _ln(s):
    bits = lax.bitcast_convert_type(s, jnp.int32)
    y = bits.astype(jnp.float32) * (_LN2 / 8388608.0) - ((127.0 - 0.0450466) * _LN2)
    for _ in range(2):
        y = y + s * jnp.exp(-y) - 1.0
    return y


def _sc_body(score_hbm, target_hbm, out_hbm, sbuf, tbuf, obuf,
             ss0, ss1, st0, st1):
    cid = lax.axis_index("c")
    sid = lax.axis_index("s")
    wid = sid * 2 + cid
    b = wid // _WPB
    woff = (wid % _WPB) * _PPW
    sems = ((ss0, st0), (ss1, st1))

    def start(j, par):
        off = woff + j * _F
        pltpu.async_copy(score_hbm.at[b, :, pl.ds(off, _F)], sbuf.at[par],
                         sems[par][0])
        pltpu.async_copy(target_hbm.at[b, pl.ds(off, _F)], tbuf.at[par],
                         sems[par][1])

    def wait(par):
        pltpu.make_async_copy(score_hbm.at[b, :, pl.ds(woff, _F)],
                              sbuf.at[par], sems[par][0]).wait()
        pltpu.make_async_copy(target_hbm.at[b, pl.ds(woff, _F)],
                              tbuf.at[par], sems[par][1]).wait()

    def make_group(par):
        sref = sbuf.at[par]

        def group(i, accs):
            nll, cnt, saa, sbb, scc = accs
            base = i * 16
            t = tbuf[par, pl.ds(base, 16)]
            xs = [sbuf[par, c, pl.ds(base, 16)] for c in range(_C)]
            m = xs[0]
            am = jnp.zeros((16,), jnp.float32)
            for c in range(1, _C):
                gt = xs[c] > m
                m = jnp.where(gt, xs[c], m)
                am = jnp.where(gt, jnp.float32(c), am)
            s = jnp.exp(xs[0] - m)
            for c in range(1, _C):
                s = s + jnp.exp(xs[c] - m)
            pidx = base + lax.iota(jnp.int32, 16)
            t0 = jnp.maximum(t, 0)
            xt = plsc.load_gather(sref, [t0, pidx])
            lse = _ln(s) + m
            valid = t >= 0
            vf = jnp.where(valid, 1.0, 0.0).astype(jnp.float32)
            tf = t.astype(jnp.float32)
            nll = nll + jnp.where(valid, lse - xt, 0.0)
            cnt = cnt + vf
            saa = saa + am * tf
            sbb = sbb + am * am
            scc = scc + tf * tf
            return (nll, cnt, saa, sbb, scc)

        return group

    start(0, 0)
    start(1, 1)

    def pair(j2, accs):
        j = j2 * 2
        for par in range(2):
            jj = j + par
            wait(par)
            accs = lax.fori_loop(0, _G, make_group(par), accs, unroll=False)
            start(jnp.minimum(jj + 2, _NCH - 1), par)
        return accs

    zeros = jnp.zeros((16,), jnp.float32)
    accs = lax.fori_loop(0, _NCH // 2, pair,
                         (zeros, zeros, zeros, zeros, zeros), unroll=False)
    wait(0)
    wait(1)
    for q in range(5):
        obuf[q, :] = accs[q]
    pltpu.sync_copy(obuf, out_hbm.at[wid])


@jax.jit
def _run(score3, target2):
    mesh = plsc.VectorSubcoreMesh(core_axis_name="c", subcore_axis_name="s")
    call = pl.kernel(
        _sc_body,
        out_type=jax.ShapeDtypeStruct((_NW, 5, 16), jnp.float32),
        mesh=mesh,
        scratch_types=[
            pltpu.VMEM((2, _C, _F), jnp.float32),
            pltpu.VMEM((2, _F), jnp.int32),
            pltpu.VMEM((5, 16), jnp.float32),
            pltpu.SemaphoreType.DMA,
            pltpu.SemaphoreType.DMA,
            pltpu.SemaphoreType.DMA,
            pltpu.SemaphoreType.DMA,
        ],
        compiler_params=pltpu.CompilerParams(needs_layout_passes=False),
    )
    part = call(score3, target2)
    part = part.sum(axis=2)
    per_b = part.reshape(_B, _WPB, 5).sum(axis=1)
    nll_tot = per_b[:, 0].sum()
    cnt_tot = per_b[:, 1].sum()
    ce = nll_tot / jnp.maximum(cnt_tot, 1.0)
    a = per_b[:, 2]
    bb = per_b[:, 3] + _EPS
    cc = per_b[:, 4] + _EPS
    dice = 1.0 - 2.0 * a / (bb + cc)
    return ce + dice


def kernel(score, target, epoch):
    score3 = score.reshape(_B, _C, _HW)
    target2 = target.reshape(_B, _HW)
    return _run(score3, target2)


# speedup vs baseline: 3.0870x; 2.0100x over previous
import functools

import jax
import jax.numpy as jnp
from jax import lax
from jax.experimental import pallas as pl
from jax.experimental.pallas import tpu as pltpu
from jax.experimental.pallas import tpu_sc as plsc

_B, _C, _H, _W = 4, 19, 512, 512
_NW = 32
_WPB = _NW // _B
_RPW = _H // _WPB
_RPC = 2
_NCH = _RPW // _RPC
_G = (_RPC * _W) // 16

_LN2 = 0.6931471805599453
_EPS = 1e-3


def _ln(s):
    bits = lax.bitcast_convert_type(s, jnp.int32)
    y = bits.astype(jnp.float32) * (_LN2 / 8388608.0) - ((127.0 - 0.0450466) * _LN2)
    for _ in range(2):
        y = y + s * jnp.exp(-y) - 1.0
    return y


def _sc_body(score_hbm, target_hbm, out_hbm, sbuf, tbuf, obuf,
             ss0, ss1, st0, st1):
    cid = lax.axis_index("c")
    sid = lax.axis_index("s")
    wid = sid * 2 + cid
    b = wid // _WPB
    wrow = (wid % _WPB) * _RPW
    sems = ((ss0, st0), (ss1, st1))

    def start(j, par):
        r0 = wrow + j * _RPC
        pltpu.async_copy(score_hbm.at[b, :, pl.ds(r0, _RPC), :],
                         sbuf.at[par], sems[par][0])
        pltpu.async_copy(target_hbm.at[b, pl.ds(r0, _RPC), :],
                         tbuf.at[par], sems[par][1])

    def wait(par):
        pltpu.make_async_copy(score_hbm.at[b, :, pl.ds(wrow, _RPC), :],
                              sbuf.at[par], sems[par][0]).wait()
        pltpu.make_async_copy(target_hbm.at[b, pl.ds(wrow, _RPC), :],
                              tbuf.at[par], sems[par][1]).wait()

    def make_group(par):
        sref = sbuf.at[par]

        def group(i, accs):
            nll, cnt, saa, sbb, scc = accs
            r = lax.shift_right_logical(i, 5)
            col = lax.shift_left(jnp.bitwise_and(i, 31), 4)
            t = tbuf[par, r, pl.ds(col, 16)]
            xs = [sbuf[par, c, r, pl.ds(col, 16)] for c in range(_C)]
            m = xs[0]
            am = jnp.zeros((16,), jnp.float32)
            for c in range(1, _C):
                gt = xs[c] > m
                m = jnp.where(gt, xs[c], m)
                am = jnp.where(gt, jnp.float32(c), am)
            s = jnp.exp(xs[0] - m)
            for c in range(1, _C):
                s = s + jnp.exp(xs[c] - m)
            rvec = jnp.full((16,), r, jnp.int32)
            cvec = col + lax.iota(jnp.int32, 16)
            t0 = jnp.maximum(t, 0)
            xt = plsc.load_gather(sref, [t0, rvec, cvec])
            lse = _ln(s) + m
            valid = t >= 0
            vf = jnp.where(valid, 1.0, 0.0).astype(jnp.float32)
            tf = t.astype(jnp.float32)
            nll = nll + jnp.where(valid, lse - xt, 0.0)
            cnt = cnt + vf
            saa = saa + am * tf
            sbb = sbb + am * am
            scc = scc + tf * tf
            return (nll, cnt, saa, sbb, scc)

        return group

    start(0, 0)
    start(1, 1)

    def pair(j2, accs):
        j = j2 * 2
        for par in range(2):
            jj = j + par
            wait(par)
            accs = lax.fori_loop(0, _G, make_group(par), accs, unroll=False)
            start(jnp.minimum(jj + 2, _NCH - 1), par)
        return accs

    zeros = jnp.zeros((16,), jnp.float32)
    accs = lax.fori_loop(0, _NCH // 2, pair,
                         (zeros, zeros, zeros, zeros, zeros), unroll=False)
    wait(0)
    wait(1)
    for q in range(5):
        obuf[q, :] = accs[q]
    pltpu.sync_copy(obuf, out_hbm.at[wid])


@jax.jit
def _run(score, target):
    mesh = plsc.VectorSubcoreMesh(core_axis_name="c", subcore_axis_name="s")
    call = pl.kernel(
        _sc_body,
        out_type=jax.ShapeDtypeStruct((_NW, 5, 16), jnp.float32),
        mesh=mesh,
        scratch_types=[
            pltpu.VMEM((2, _C, _RPC, _W), jnp.float32),
            pltpu.VMEM((2, _RPC, _W), jnp.int32),
            pltpu.VMEM((5, 16), jnp.float32),
            pltpu.SemaphoreType.DMA,
            pltpu.SemaphoreType.DMA,
            pltpu.SemaphoreType.DMA,
            pltpu.SemaphoreType.DMA,
        ],
        compiler_params=pltpu.CompilerParams(needs_layout_passes=False),
    )
    part = call(score, target)
    part = part.sum(axis=2)
    per_b = part.reshape(_B, _WPB, 5).sum(axis=1)
    nll_tot = per_b[:, 0].sum()
    cnt_tot = per_b[:, 1].sum()
    ce = nll_tot / jnp.maximum(cnt_tot, 1.0)
    a = per_b[:, 2]
    bb = per_b[:, 3] + _EPS
    cc = per_b[:, 4] + _EPS
    dice = 1.0 - 2.0 * a / (bb + cc)
    return ce + dice


def kernel(score, target, epoch):
    return _run(score, target)
